# baseline (speedup 1.0000x reference)
import jax
import jax.numpy as jnp
from jax.experimental import pallas as pl
from jax.experimental.pallas import tpu as pltpu

_SMOOTH = 1.0
_NC = 19


def _dice_tc_body(pred_ref, tgt_ref, out_ref, acc_ref):
    c = pl.program_id(1)

    pred = pred_ref[0, 0]
    tgt = tgt_ref[0]
    mask = tgt == c
    nom = jnp.sum(jnp.where(mask, pred, 0.0))
    isum = jnp.sum(pred)
    tsum = jnp.sum(jnp.where(mask, 1.0, 0.0))
    frac = (2.0 * nom + _SMOOTH) / (isum + tsum + _SMOOTH)

    total = jnp.where(c == 0, 0.0, acc_ref[0]) + frac
    acc_ref[0] = total

    @pl.when(c == _NC - 1)
    def _finish():
        out_ref[0, 0, :] = jnp.full((128,), -total / _NC, dtype=jnp.float32)


def kernel(predictions, targets):
    out = pl.pallas_call(
        _dice_tc_body,
        grid=(8, _NC),
        in_specs=[
            pl.BlockSpec((1, 1, 512, 512), lambda b, c: (b, c, 0, 0)),
            pl.BlockSpec((1, 512, 512), lambda b, c: (b, 0, 0)),
        ],
        out_specs=pl.BlockSpec((1, 1, 128), lambda b, c: (b, 0, 0)),
        out_shape=jax.ShapeDtypeStruct((8, 1, 128), jnp.float32),
        scratch_shapes=[
            pltpu.SMEM((1,), jnp.float32),
        ],
    )(predictions, targets)
    return out[:, 0, 0]


# speedup vs baseline: 1.9215x; 1.9215x over previous
import jax
import jax.numpy as jnp
from jax.experimental import pallas as pl
from jax.experimental.pallas import tpu as pltpu

_SMOOTH = 1.0
_HT = 128
_NS = 512 // _HT
_NC = 19


def _dice_tc_body(pred_ref, tgt_ref, out_ref, nom_acc, isum_acc, tsum_acc):
    s = pl.program_id(1)

    @pl.when(s == 0)
    def _init():
        nom_acc[...] = jnp.zeros_like(nom_acc)
        isum_acc[...] = jnp.zeros_like(isum_acc)
        tsum_acc[...] = jnp.zeros_like(tsum_acc)

    tgt = tgt_ref[0]
    for c in range(_NC):
        pred_c = pred_ref[0, c]
        mask = tgt == c
        nom_p = jnp.where(mask, pred_c, 0.0).reshape(_HT // 8, 8, 512).sum(axis=0)
        isum_p = pred_c.reshape(_HT // 8, 8, 512).sum(axis=0)
        tsum_p = jnp.where(mask, 1.0, 0.0).reshape(_HT // 8, 8, 512).sum(axis=0)
        nom_acc[c] += nom_p
        isum_acc[c] += isum_p
        tsum_acc[c] += tsum_p

    @pl.when(s == _NS - 1)
    def _finish():
        nom = jnp.sum(nom_acc[...], axis=(1, 2))
        isum = jnp.sum(isum_acc[...], axis=(1, 2))
        tsum = jnp.sum(tsum_acc[...], axis=(1, 2))
        frac = (2.0 * nom + _SMOOTH) / (isum + tsum + _SMOOTH)
        loss = -jnp.sum(frac) / _NC
        out_ref[0, 0, :] = jnp.full((128,), loss, dtype=jnp.float32)


def kernel(predictions, targets):
    out = pl.pallas_call(
        _dice_tc_body,
        grid=(8, _NS),
        in_specs=[
            pl.BlockSpec((1, _NC, _HT, 512), lambda b, s: (b, 0, s, 0)),
            pl.BlockSpec((1, _HT, 512), lambda b, s: (b, s, 0)),
        ],
        out_specs=pl.BlockSpec((1, 1, 128), lambda b, s: (b, 0, 0)),
        out_shape=jax.ShapeDtypeStruct((8, 1, 128), jnp.float32),
        scratch_shapes=[
            pltpu.VMEM((_NC, 8, 512), jnp.float32),
            pltpu.VMEM((_NC, 8, 512), jnp.float32),
            pltpu.VMEM((_NC, 8, 512), jnp.float32),
        ],
    )(predictions, targets)
    return out[:, 0, 0]


# speedup vs baseline: 1.9879x; 1.0345x over previous
import jax
import jax.numpy as jnp
from jax.experimental import pallas as pl
from jax.experimental.pallas import tpu as pltpu

_SMOOTH = 1.0
_HT = 256
_NS = 512 // _HT
_NC = 19


def _dice_tc_body(pred_ref, tgt_ref, out_ref, nom_acc, isum_acc, tsum_acc):
    s = pl.program_id(1)

    @pl.when(s == 0)
    def _init():
        nom_acc[...] = jnp.zeros_like(nom_acc)
        isum_acc[...] = jnp.zeros_like(isum_acc)
        tsum_acc[...] = jnp.zeros_like(tsum_acc)

    tgt = tgt_ref[0]
    for c in range(_NC):
        pred_c = pred_ref[0, c]
        mask = tgt == c
        nom_p = jnp.where(mask, pred_c, 0.0).reshape(_HT // 8, 8, 512).sum(axis=0)
        isum_p = pred_c.reshape(_HT // 8, 8, 512).sum(axis=0)
        tsum_p = jnp.where(mask, 1.0, 0.0).reshape(_HT // 8, 8, 512).sum(axis=0)
        nom_acc[c] += nom_p
        isum_acc[c] += isum_p
        tsum_acc[c] += tsum_p

    @pl.when(s == _NS - 1)
    def _finish():
        nom = jnp.sum(nom_acc[...], axis=(1, 2))
        isum = jnp.sum(isum_acc[...], axis=(1, 2))
        tsum = jnp.sum(tsum_acc[...], axis=(1, 2))
        frac = (2.0 * nom + _SMOOTH) / (isum + tsum + _SMOOTH)
        loss = -jnp.sum(frac) / _NC
        out_ref[0, 0, :] = jnp.full((128,), loss, dtype=jnp.float32)


def kernel(predictions, targets):
    out = pl.pallas_call(
        _dice_tc_body,
        grid=(8, _NS),
        in_specs=[
            pl.BlockSpec((1, _NC, _HT, 512), lambda b, s: (b, 0, s, 0)),
            pl.BlockSpec((1, _HT, 512), lambda b, s: (b, s, 0)),
        ],
        out_specs=pl.BlockSpec((1, 1, 128), lambda b, s: (b, 0, 0)),
        out_shape=jax.ShapeDtypeStruct((8, 1, 128), jnp.float32),
        scratch_shapes=[
            pltpu.VMEM((_NC, 8, 512), jnp.float32),
            pltpu.VMEM((_NC, 8, 512), jnp.float32),
            pltpu.VMEM((_NC, 8, 512), jnp.float32),
        ],
    )(predictions, targets)
    return out[:, 0, 0]
